# baseline (speedup 1.0000x reference)
import functools

import jax
import jax.numpy as jnp
from jax import lax
from jax.experimental import pallas as pl
from jax.experimental.pallas import tpu as pltpu
from jax.experimental.pallas import tpu_sc as plsc

S = 2048
D = 1024
INNER = 256
E = 64
K = 8
CAP = 512
TOK = 256
NC = 2
NSUB = 16
NW = NC * NSUB
TPW = S // NW


HIMASK = -65536


def _pack_halves(vbf):
    n = vbf.shape[1] // 2
    af = lax.bitcast_convert_type(vbf[:, :n].astype(jnp.float32), jnp.int32)
    bf = lax.bitcast_convert_type(vbf[:, n:].astype(jnp.float32), jnp.int32)
    return lax.shift_right_logical(af, 16) | (bf & HIMASK)


def _unpack_halves(w):
    a = lax.bitcast_convert_type(lax.shift_left(w, 16), jnp.float32)
    b = lax.bitcast_convert_type(w & HIMASK, jnp.float32)
    return jnp.concatenate([a, b], axis=1)


def _router_body(x_ref, ct_ref, b_ref, w1_ref, b1_ref, w2_ref, b2_ref,
                 w3_ref, b3_ref, base_ref, aff_ref, gate_ref, topi_ref,
                 dsc_ref, dsct_ref, dcbt_ref, wexp_ref, xbf_ref, cnt_ref):
    pid = pl.program_id(0)

    @pl.when(pid == 0)
    def _init():
        cnt_ref[...] = jnp.zeros_like(cnt_ref)

    x = x_ref[...]
    xbf = x.astype(jnp.bfloat16)
    xbf_ref[...] = _pack_halves(xbf)
    aff = jax.nn.sigmoid(
        jnp.dot(x, ct_ref[...], preferred_element_type=jnp.float32))
    aff_ref[...] = aff
    cur = aff + b_ref[...]
    lanes = lax.broadcasted_iota(jnp.int32, (TOK, E), 1)
    ams, avals, onehots = [], [], []
    for _ in range(K):
        mx = jnp.max(cur, axis=1, keepdims=True)
        am = jnp.min(jnp.where(cur == mx, lanes, E), axis=1, keepdims=True)
        oh = lanes == am
        avals.append(jnp.sum(jnp.where(oh, aff, 0.0), axis=1, keepdims=True))
        ams.append(am)
        onehots.append(oh)
        cur = jnp.where(oh, -jnp.inf, cur)
    topi = jnp.concatenate(ams, axis=1)
    sel = jnp.concatenate(avals, axis=1)
    mx = jnp.max(sel, axis=1, keepdims=True)
    ex = jnp.exp(sel - mx)
    gate = ex / jnp.sum(ex, axis=1, keepdims=True)
    topi_ref[...] = topi
    gate_ref[...] = gate

    usage = jnp.zeros((TOK, E), jnp.int32)
    for oh in onehots:
        usage = usage + oh.astype(jnp.int32)
    incl = usage
    shift = 1
    while shift < TOK:
        incl = incl + jnp.concatenate(
            [jnp.zeros((shift, E), jnp.int32), incl[:TOK - shift]], axis=0)
        shift *= 2
    basecnt = (incl - usage) + cnt_ref[0:1, :]
    dsc_cols, wc_cols = [], []
    for k in range(K):
        slot = jnp.sum(jnp.where(onehots[k], basecnt, 0), axis=1,
                       keepdims=True)
        dest = ams[k] * CAP + slot
        valid = slot < CAP
        dsc_cols.append(jnp.where(valid, dest, E * CAP))
        wc_cols.append(jnp.where(valid, gate[:, k:k + 1], 0.0))
    dsc = jnp.concatenate(dsc_cols, axis=1)
    dsc_ref[...] = dsc
    dsct = jnp.transpose(dsc)
    dsct_ref[...] = dsct
    dcbt_ref[...] = dsct
    wexp_ref[...] = jnp.broadcast_to(
        jnp.concatenate(wc_cols, axis=1)[:, :, None], (TOK, K, 128))
    cnt_ref[0:1, :] = cnt_ref[0:1, :] + incl[TOK - 1:TOK, :]

    xb = x.astype(jnp.bfloat16)
    a = jnp.dot(xb, w1_ref[...],
                preferred_element_type=jnp.float32) + b1_ref[...]
    u = jnp.dot(xb, w3_ref[...],
                preferred_element_type=jnp.float32) + b3_ref[...]
    h = ((a * jax.nn.sigmoid(a)) * u).astype(jnp.bfloat16)
    sh = jnp.dot(h, w2_ref[...],
                 preferred_element_type=jnp.float32) + b2_ref[...]
    base_ref[...] = x + sh


def _run_router(x2d, ct, b2d, w1, b1, w2, b2, w3, b3):
    return pl.pallas_call(
        _router_body,
        grid=(S // TOK,),
        in_specs=[
            pl.BlockSpec((TOK, D), lambda i: (i, 0)),
            pl.BlockSpec((D, E), lambda i: (0, 0)),
            pl.BlockSpec((1, E), lambda i: (0, 0)),
            pl.BlockSpec((D, INNER), lambda i: (0, 0)),
            pl.BlockSpec((1, INNER), lambda i: (0, 0)),
            pl.BlockSpec((INNER, D), lambda i: (0, 0)),
            pl.BlockSpec((1, D), lambda i: (0, 0)),
            pl.BlockSpec((D, INNER), lambda i: (0, 0)),
            pl.BlockSpec((1, INNER), lambda i: (0, 0)),
        ],
        out_specs=[
            pl.BlockSpec((TOK, D), lambda i: (i, 0)),
            pl.BlockSpec((TOK, E), lambda i: (i, 0)),
            pl.BlockSpec((TOK, K), lambda i: (i, 0)),
            pl.BlockSpec((TOK, K), lambda i: (i, 0)),
            pl.BlockSpec((TOK, K), lambda i: (i, 0)),
            pl.BlockSpec((K, TOK), lambda i: (0, i)),
            pl.BlockSpec((K, TOK), lambda i: (0, i)),
            pl.BlockSpec((TOK, K, 128), lambda i: (i, 0, 0)),
            pl.BlockSpec((TOK, D // 2), lambda i: (i, 0)),
        ],
        out_shape=[
            jax.ShapeDtypeStruct((S, D), jnp.float32),
            jax.ShapeDtypeStruct((S, E), jnp.float32),
            jax.ShapeDtypeStruct((S, K), jnp.float32),
            jax.ShapeDtypeStruct((S, K), jnp.int32),
            jax.ShapeDtypeStruct((S, K), jnp.int32),
            jax.ShapeDtypeStruct((K, S), jnp.int32),
            jax.ShapeDtypeStruct((K, S), jnp.int32),
            jax.ShapeDtypeStruct((S, K, 128), jnp.float32),
            jax.ShapeDtypeStruct((S, D // 2), jnp.int32),
        ],
        scratch_shapes=[pltpu.VMEM((8, E), jnp.int32)],
    )(x2d, ct, b2d, w1, b1, w2, b2, w3, b3)


def _ffn_body(xd_ref, wd_ref, w1_ref, b1_ref, w2_ref, b2_ref, w3_ref, b3_ref,
              y_ref):
    pid = pl.program_id(0)

    @pl.when(pid < E)
    def _compute():
        x = _unpack_halves(xd_ref[...]).astype(jnp.bfloat16)
        a = jnp.dot(x, w1_ref[0],
                    preferred_element_type=jnp.float32) + b1_ref[0]
        u = jnp.dot(x, w3_ref[0],
                    preferred_element_type=jnp.float32) + b3_ref[0]
        h = ((a * jax.nn.sigmoid(a)) * u).astype(jnp.bfloat16)
        y = jnp.dot(h, w2_ref[0],
                    preferred_element_type=jnp.float32) + b2_ref[0]
        y_ref[...] = y * wd_ref[...][:, 0:1]

    @pl.when(pid == E)
    def _zero_pad():
        y_ref[...] = jnp.zeros_like(y_ref)


def _run_ffn(disp, wdisp, rW1b, rb1, rW2b, rb2, rW3b, rb3):
    def clamped(*unit):
        def index_map(e):
            return (jnp.minimum(e, E - 1),) + unit
        return index_map

    return pl.pallas_call(
        _ffn_body,
        grid=(E + 1,),
        in_specs=[
            pl.BlockSpec((CAP, D // 2), clamped(0)),
            pl.BlockSpec((CAP, 128), clamped(0)),
            pl.BlockSpec((1, D, INNER), clamped(0, 0)),
            pl.BlockSpec((1, 1, INNER), clamped(0, 0)),
            pl.BlockSpec((1, INNER, D), clamped(0, 0)),
            pl.BlockSpec((1, 1, D), clamped(0, 0)),
            pl.BlockSpec((1, D, INNER), clamped(0, 0)),
            pl.BlockSpec((1, 1, INNER), clamped(0, 0)),
        ],
        out_specs=pl.BlockSpec((CAP, D), lambda e: (e, 0)),
        out_shape=jax.ShapeDtypeStruct((E * CAP + CAP, D), jnp.float32),
    )(disp, wdisp, rW1b, rb1.reshape(E, 1, INNER), rW2b,
      rb2.reshape(E, 1, D), rW3b, rb3.reshape(E, 1, INNER))


def _run_dispatch(xbf, dsc_t, dsc_flat, wexp_flat):
    mesh = plsc.VectorSubcoreMesh(core_axis_name="c", subcore_axis_name="s")
    chunk = 32
    pch = chunk * K

    @functools.partial(
        pl.kernel,
        mesh=mesh,
        out_type=[jax.ShapeDtypeStruct((E * CAP + 8, D // 2), jnp.int32),
                  jax.ShapeDtypeStruct((E * CAP + 8, 128), jnp.float32)],
        scratch_types=(
            [pltpu.VMEM((chunk, D // 2), jnp.int32),
             pltpu.VMEM((pch, 128), jnp.float32),
             pltpu.VMEM((pch,), jnp.int32)]
            + [pltpu.VMEM((chunk,), jnp.int32) for _ in range(K)]
            + [pltpu.SemaphoreType.DMA]
        ),
    )
    def body(x_hbm, dsct_hbm, dscf_hbm, wexpf_hbm, disp_hbm, wdisp_hbm,
             x_v, w_v, idxw_v, *rest):
        idx_vs, sem = rest[:K], rest[K]
        wid = lax.axis_index("s") * NC + lax.axis_index("c")
        for c in range(TPW // chunk):
            t0 = wid * TPW + c * chunk
            p0 = t0 * K
            cps = [pltpu.async_copy(x_hbm.at[pl.ds(t0, chunk)], x_v, sem),
                   pltpu.async_copy(dscf_hbm.at[pl.ds(p0, pch)], idxw_v, sem),
                   pltpu.async_copy(wexpf_hbm.at[pl.ds(p0, pch)], w_v, sem)]
            cps += [pltpu.async_copy(dsct_hbm.at[k, pl.ds(t0, chunk)],
                                     idx_vs[k], sem) for k in range(K)]
            for cp in cps:
                cp.wait()
            cps = [pltpu.async_copy(x_v, disp_hbm.at[idx_vs[k]], sem)
                   for k in range(K)]
            cps.append(pltpu.async_copy(w_v, wdisp_hbm.at[idxw_v], sem))
            for cp in cps:
                cp.wait()

    return body(xbf, dsc_t, dsc_flat, wexp_flat)


def _run_combine(base_flat, ybuf, dcb_t):
    mesh = plsc.VectorSubcoreMesh(core_axis_name="c", subcore_axis_name="s")
    T = 8

    @functools.partial(
        pl.kernel,
        mesh=mesh,
        out_type=jax.ShapeDtypeStruct((S * D,), jnp.float32),
        scratch_types=(
            [pltpu.VMEM((K * T, D), jnp.float32),
             pltpu.VMEM((T * D,), jnp.float32),
             pltpu.VMEM((T * D,), jnp.float32)]
            + [pltpu.VMEM((T,), jnp.int32) for _ in range(K)]
            + [pltpu.SemaphoreType.DMA]
        ),
    )
    def body(base_hbm, y_hbm, dcb_hbm, hid_hbm, g_v, b_v, o_v, *rest):
        idx_vs, sem = rest[:K], rest[K]
        wid = lax.axis_index("s") * NC + lax.axis_index("c")

        def chunk_body(c, carry):
            t0 = wid * TPW + c * T
            cps = [pltpu.async_copy(base_hbm.at[pl.ds(t0 * D, T * D)], b_v,
                                    sem)]
            cps += [pltpu.async_copy(dcb_hbm.at[k, pl.ds(t0, T)], idx_vs[k],
                                     sem) for k in range(K)]
            for cp in cps:
                cp.wait()
            cps = [pltpu.async_copy(y_hbm.at[idx_vs[k]],
                                    g_v.at[pl.ds(k * T, T)], sem)
                   for k in range(K)]
            for cp in cps:
                cp.wait()
            for t in range(T):

                def vec_body(j, inner, _t=t):
                    sl = pl.ds(j * 16, 16)
                    acc = b_v[pl.ds(_t * D + j * 16, 16)]
                    for k in range(K):
                        acc = acc + g_v[k * T + _t, sl]
                    o_v[pl.ds(_t * D + j * 16, 16)] = acc
                    return inner

                lax.fori_loop(0, D // 16, vec_body, 0)
            pltpu.sync_copy(o_v, hid_hbm.at[pl.ds(t0 * D, T * D)])
            return carry

        lax.fori_loop(0, TPW // T, chunk_body, 0)

    return body(base_flat, ybuf, dcb_t)


def kernel(input_embeddings, centroids, biases, sW1, sb1, sW2, sb2, sW3, sb3,
           rW1, rb1, rW2, rb2, rW3, rb3):
    x2d = input_embeddings.reshape(S, D)
    base, aff, gate, topi, dsc, dsc_t, dcb_t, wexp, xbf = _run_router(
        x2d, centroids.T, biases.reshape(1, E),
        sW1[0].astype(jnp.bfloat16), sb1, sW2[0].astype(jnp.bfloat16), sb2,
        sW3[0].astype(jnp.bfloat16), sb3)
    disp, wdisp = _run_dispatch(xbf, dsc_t, dsc.reshape(S * K),
                                wexp.reshape(S * K, 128))
    ybuf = _run_ffn(disp, wdisp, rW1.astype(jnp.bfloat16), rb1,
                    rW2.astype(jnp.bfloat16), rb2,
                    rW3.astype(jnp.bfloat16), rb3)
    hid = _run_combine(base.reshape(S * D), ybuf, dcb_t)
    return (hid.reshape(1, S, D), aff.reshape(1, S, E),
            gate.reshape(1, S, K), topi.reshape(1, S, K))
